# baseline (speedup 1.0000x reference)
import dataclasses
import functools

import jax
import jax.numpy as jnp
from jax import lax
from jax.experimental import pallas as pl
from jax.experimental.pallas import tpu as pltpu
from jax.experimental.pallas import tpu_sc as plsc

BI = 1024


CH = 4


def _stats_body(a_ref, bt_ref, alpha_ref, beta_ref, cm_ref, cs_ref):
    i = pl.program_id(1)
    a = a_ref[0]
    nb = bt_ref.shape[2]
    cw = nb // CH
    @pl.when(i == 0)
    def _():
        cm_ref[...] = jnp.full(cm_ref.shape, -jnp.inf, jnp.float32)
        cs_ref[...] = jnp.zeros(cs_ref.shape, jnp.float32)
    ms, ss = [], []
    for c in range(CH):
        sl = slice(c * cw, (c + 1) * cw)
        corr = lax.dot_general(a, bt_ref[0, :, sl], (((1,), (0,)), ((), ())),
                               preferred_element_type=jnp.float32)
        m = jnp.max(corr, axis=1, keepdims=True)
        ms.append(m)
        ss.append(jnp.sum(jnp.exp(corr - m), axis=1, keepdims=True))
        tm = jnp.max(corr, axis=0, keepdims=True)
        m_old = cm_ref[:, sl]
        m_new = jnp.maximum(m_old, tm)
        cs_ref[:, sl] = (cs_ref[:, sl] * jnp.exp(m_old - m_new)
                         + jnp.sum(jnp.exp(corr - m_new), axis=0,
                                   keepdims=True))
        cm_ref[:, sl] = m_new
        beta_ref[0, :, sl] = m_new + jnp.log(cs_ref[:, sl])
    mrow, srow = ms[0], ss[0]
    for c in range(1, CH):
        m = jnp.maximum(mrow, ms[c])
        srow = srow * jnp.exp(mrow - m) + ss[c] * jnp.exp(ms[c] - m)
        mrow = m
    alpha_ref[0] = mrow + jnp.log(srow)


def _argmax_body(a_ref, bt_ref, ah_ref, bh_ref, g_ref,
                 ridx_ref, rval_ref, cidx_ref, am_ref, ai_ref):
    i = pl.program_id(1)
    a = a_ref[0]
    ah = ah_ref[0]
    @pl.when(i == 0)
    def _():
        am_ref[...] = jnp.full(am_ref.shape, -jnp.inf, jnp.float32)
        ai_ref[...] = jnp.zeros(ai_ref.shape, jnp.int32)
    corr = lax.dot_general(a, bt_ref[0], (((1,), (0,)), ((), ())),
                           preferred_element_type=jnp.float32)
    w = corr - bh_ref[0]
    rmax = jnp.max(w, axis=1, keepdims=True)
    ridx_ref[0] = jnp.argmax(w, axis=1).astype(jnp.int32)[:, None]
    rval_ref[0] = jnp.exp(rmax + rmax - g_ref[0])
    v = w - ah
    vm = jnp.max(v, axis=0, keepdims=True)
    vidx = (jnp.argmax(v, axis=0).astype(jnp.int32) + i * BI)[None, :]
    upd = vm > am_ref[...]
    am_ref[...] = jnp.where(upd, vm, am_ref[...])
    ai_ref[...] = jnp.where(upd, vidx, ai_ref[...])
    cidx_ref[0] = ai_ref[...]


def _tc_passes(desc_a, desc_b):
    b, na, d = desc_a.shape
    nb = desc_b.shape[1]
    ni = na // BI
    a3 = desc_a.astype(jnp.bfloat16)
    bt = desc_b.astype(jnp.bfloat16).transpose(0, 2, 1)

    a_spec = pl.BlockSpec((1, BI, d), lambda bb, ii: (bb, ii, 0))
    bt_spec = pl.BlockSpec((1, d, nb), lambda bb, ii: (bb, 0, 0))
    col_spec = pl.BlockSpec((1, BI, 1), lambda bb, ii: (bb * ni + ii, 0, 0))
    row_spec = pl.BlockSpec((1, 1, nb), lambda bb, ii: (bb, 0, 0))

    alpha, beta = pl.pallas_call(
        _stats_body,
        grid=(b, ni),
        in_specs=[a_spec, bt_spec],
        out_specs=[col_spec, row_spec],
        out_shape=[jax.ShapeDtypeStruct((b * ni, BI, 1), jnp.float32),
                   jax.ShapeDtypeStruct((b, 1, nb), jnp.float32)],
        scratch_shapes=[pltpu.VMEM((1, nb), jnp.float32),
                        pltpu.VMEM((1, nb), jnp.float32)],
    )(a3, bt)

    amean = jnp.mean(alpha.reshape(b, ni * BI), axis=1).reshape(b, 1, 1)
    bmean = jnp.mean(beta, axis=2, keepdims=True)
    ah = ((alpha.reshape(b, ni, BI, 1) - amean[:, None]) * 0.5
          ).reshape(b * ni, BI, 1)
    bh = (beta - bmean) * 0.5
    gamma = (alpha.reshape(b, ni, BI, 1) + bmean[:, None]).reshape(b * ni, BI, 1)

    ridx, rval, cidx = pl.pallas_call(
        _argmax_body,
        grid=(b, ni),
        in_specs=[a_spec, bt_spec, col_spec, row_spec, col_spec],
        out_specs=[col_spec, col_spec, row_spec],
        out_shape=[jax.ShapeDtypeStruct((b * ni, BI, 1), jnp.int32),
                   jax.ShapeDtypeStruct((b * ni, BI, 1), jnp.float32),
                   jax.ShapeDtypeStruct((b, 1, nb), jnp.int32)],
        scratch_shapes=[pltpu.VMEM((1, nb), jnp.float32),
                        pltpu.VMEM((1, nb), jnp.int32)],
    )(a3, bt, ah, bh, gamma)

    max_idx_a = ridx.reshape(b, na)
    max_vals_a = rval.reshape(b, na)
    max_idx_b = cidx.reshape(b, nb)
    return max_idx_a, max_vals_a, max_idx_b


def _sc_epilogue(idx_a, vals_a, idx_b):
    b, na = idx_a.shape
    nb = idx_b.shape[1]
    info = plsc.get_sparse_core_info()
    nw = info.num_cores * info.num_subcores
    chunk = na // nw
    mesh = plsc.VectorSubcoreMesh(core_axis_name="c", subcore_axis_name="s")
    cp = pltpu.CompilerParams()
    if "needs_layout_passes" in pltpu.CompilerParams.__dataclass_fields__:
        cp = dataclasses.replace(cp, needs_layout_passes=False)

    @functools.partial(
        pl.kernel, mesh=mesh, compiler_params=cp,
        out_type=[jax.ShapeDtypeStruct((b, na), jnp.int32),
                  jax.ShapeDtypeStruct((b, nb), jnp.int32),
                  jax.ShapeDtypeStruct((b, na), jnp.float32),
                  jax.ShapeDtypeStruct((b, nb), jnp.float32)],
        scratch_types=[pltpu.VMEM((b * na,), jnp.int32),
                       pltpu.VMEM((b * nb,), jnp.int32),
                       pltpu.VMEM((b * na,), jnp.float32),
                       pltpu.VMEM((b, chunk), jnp.int32),
                       pltpu.VMEM((b, chunk), jnp.float32),
                       pltpu.VMEM((b, chunk), jnp.int32),
                       pltpu.VMEM((b, chunk), jnp.float32),
                       pltpu.SemaphoreType.DMA((b,)),
                       pltpu.SemaphoreType.DMA((b,))],
    )
    def epilogue(ia_hbm, va_hbm, ib_hbm, m0_hbm, m1_hbm, s0_hbm, s1_hbm,
                 ia_t, ib_t, va_t, m0_c, s0_c, m1_c, s1_c, insem, outsem):
        wid = lax.axis_index("s") * info.num_cores + lax.axis_index("c")
        base = wid * chunk
        in_copies = []
        for bb in range(b):
            in_copies.append((
                pltpu.async_copy(ia_hbm.at[bb], ia_t.at[pl.ds(bb * na, na)],
                                 insem.at[bb]),
                pltpu.async_copy(ib_hbm.at[bb], ib_t.at[pl.ds(bb * nb, nb)],
                                 insem.at[bb]),
                pltpu.async_copy(va_hbm.at[bb], va_t.at[pl.ds(bb * na, na)],
                                 insem.at[bb]),
            ))
        out_copies = []
        for bb in range(b):
            for cpy in in_copies[bb]:
                cpy.wait()

            @pl.loop(0, chunk, step=16)
            def _row(v):
                ja = ia_t[pl.ds(bb * na + base + v, 16)]
                back = plsc.load_gather(ib_t, [ja + bb * nb])
                rowid = lax.iota(jnp.int32, 16) + (base + v)
                val = va_t[pl.ds(bb * na + base + v, 16)]
                ok = (back == rowid) & (val > 0.0)
                m0_c[bb, pl.ds(v, 16)] = jnp.where(ok, ja, -1)
                s0_c[bb, pl.ds(v, 16)] = jnp.where(ok, val, 0.0)

            @pl.loop(0, chunk, step=16)
            def _col(v):
                jb = ib_t[pl.ds(bb * nb + base + v, 16)]
                fwd = plsc.load_gather(ia_t, [jb + bb * na])
                vb = plsc.load_gather(va_t, [jb + bb * na])
                colid = lax.iota(jnp.int32, 16) + (base + v)
                ok = (fwd == colid) & (vb > 0.0)
                m1_c[bb, pl.ds(v, 16)] = jnp.where(ok, jb, -1)
                s1_c[bb, pl.ds(v, 16)] = jnp.where(ok, vb, 0.0)

            out_copies.extend([
                pltpu.async_copy(m0_c.at[bb], m0_hbm.at[bb, pl.ds(base, chunk)],
                                 outsem.at[bb]),
                pltpu.async_copy(s0_c.at[bb], s0_hbm.at[bb, pl.ds(base, chunk)],
                                 outsem.at[bb]),
                pltpu.async_copy(m1_c.at[bb], m1_hbm.at[bb, pl.ds(base, chunk)],
                                 outsem.at[bb]),
                pltpu.async_copy(s1_c.at[bb], s1_hbm.at[bb, pl.ds(base, chunk)],
                                 outsem.at[bb]),
            ])
        for cpy in out_copies:
            cpy.wait()

    return epilogue(idx_a, vals_a, idx_b)


def kernel(keypoints_A, descriptions_A, keypoints_B, descriptions_B):
    max_idx_a, max_vals_a, max_idx_b = _tc_passes(descriptions_A, descriptions_B)
    matches0, matches1, scores0, scores1 = _sc_epilogue(
        max_idx_a, max_vals_a, max_idx_b)
    return matches0, matches1, scores0, scores1


# speedup vs baseline: 1.0399x; 1.0399x over previous
import dataclasses
import functools

import jax
import jax.numpy as jnp
from jax import lax
from jax.experimental import pallas as pl
from jax.experimental.pallas import tpu as pltpu
from jax.experimental.pallas import tpu_sc as plsc

BI = 1024


CH = 4


def _stats_body(a_ref, bt_ref, alpha_ref, beta_ref, amean_ref, bmean_ref,
                cm_ref, cs_ref, as_ref):
    i = pl.program_id(1)
    ni = pl.num_programs(1)
    a = a_ref[0]
    nb = bt_ref.shape[2]
    cw = nb // CH
    @pl.when(i == 0)
    def _():
        cm_ref[...] = jnp.full(cm_ref.shape, -jnp.inf, jnp.float32)
        cs_ref[...] = jnp.zeros(cs_ref.shape, jnp.float32)
        as_ref[...] = jnp.zeros(as_ref.shape, jnp.float32)
    ms, ss = [], []
    for c in range(CH):
        sl = slice(c * cw, (c + 1) * cw)
        corr = lax.dot_general(a, bt_ref[0, :, sl], (((1,), (0,)), ((), ())),
                               preferred_element_type=jnp.float32)
        m = jnp.max(corr, axis=1, keepdims=True)
        ms.append(m)
        ss.append(jnp.sum(jnp.exp(corr - m), axis=1, keepdims=True))
        tm = jnp.max(corr, axis=0, keepdims=True)
        m_old = cm_ref[:, sl]
        m_new = jnp.maximum(m_old, tm)
        cs_ref[:, sl] = (cs_ref[:, sl] * jnp.exp(m_old - m_new)
                         + jnp.sum(jnp.exp(corr - m_new), axis=0,
                                   keepdims=True))
        cm_ref[:, sl] = m_new
        beta_ref[0, :, sl] = m_new + jnp.log(cs_ref[:, sl])
    mrow, srow = ms[0], ss[0]
    for c in range(1, CH):
        m = jnp.maximum(mrow, ms[c])
        srow = srow * jnp.exp(mrow - m) + ss[c] * jnp.exp(ms[c] - m)
        mrow = m
    alpha = mrow + jnp.log(srow)
    alpha_ref[0] = alpha
    as_ref[...] = as_ref[...] + jnp.sum(alpha, axis=0, keepdims=True)
    @pl.when(i == ni - 1)
    def _():
        na_total = ni * alpha.shape[0]
        amean_ref[0] = as_ref[...] / na_total
        bmean_ref[0] = (jnp.sum(beta_ref[0], axis=1, keepdims=True) / nb)


def _argmax_body(a_ref, bt_ref, alpha_ref, beta_ref, amean_ref, bmean_ref,
                 ridx_ref, rval_ref, cidx_ref, am_ref, ai_ref):
    i = pl.program_id(1)
    a = a_ref[0]
    amean = amean_ref[0]
    bmean = bmean_ref[0]
    alpha = alpha_ref[0]
    ah = (alpha - amean) * 0.5
    bh = (beta_ref[0] - bmean) * 0.5
    g = alpha + bmean
    @pl.when(i == 0)
    def _():
        am_ref[...] = jnp.full(am_ref.shape, -jnp.inf, jnp.float32)
        ai_ref[...] = jnp.zeros(ai_ref.shape, jnp.int32)
    corr = lax.dot_general(a, bt_ref[0], (((1,), (0,)), ((), ())),
                           preferred_element_type=jnp.float32)
    w = corr - bh
    rmax = jnp.max(w, axis=1, keepdims=True)
    ridx_ref[0] = jnp.argmax(w, axis=1).astype(jnp.int32)[:, None]
    rval_ref[0] = jnp.exp(rmax + rmax - g)
    v = w - ah
    vm = jnp.max(v, axis=0, keepdims=True)
    vidx = (jnp.argmax(v, axis=0).astype(jnp.int32) + i * BI)[None, :]
    upd = vm > am_ref[...]
    am_ref[...] = jnp.where(upd, vm, am_ref[...])
    ai_ref[...] = jnp.where(upd, vidx, ai_ref[...])
    cidx_ref[0] = ai_ref[...]


def _tc_passes(desc_a, desc_b):
    b, na, d = desc_a.shape
    nb = desc_b.shape[1]
    ni = na // BI
    a3 = desc_a.astype(jnp.bfloat16)
    bt = desc_b.astype(jnp.bfloat16).transpose(0, 2, 1)

    a_spec = pl.BlockSpec((1, BI, d), lambda bb, ii: (bb, ii, 0))
    bt_spec = pl.BlockSpec((1, d, nb), lambda bb, ii: (bb, 0, 0))
    col_spec = pl.BlockSpec((1, BI, 1), lambda bb, ii: (bb * ni + ii, 0, 0))
    row_spec = pl.BlockSpec((1, 1, nb), lambda bb, ii: (bb, 0, 0))

    mean_spec = pl.BlockSpec((1, 1, 1), lambda bb, ii: (bb, 0, 0))

    alpha, beta, amean, bmean = pl.pallas_call(
        _stats_body,
        grid=(b, ni),
        in_specs=[a_spec, bt_spec],
        out_specs=[col_spec, row_spec, mean_spec, mean_spec],
        out_shape=[jax.ShapeDtypeStruct((b * ni, BI, 1), jnp.float32),
                   jax.ShapeDtypeStruct((b, 1, nb), jnp.float32),
                   jax.ShapeDtypeStruct((b, 1, 1), jnp.float32),
                   jax.ShapeDtypeStruct((b, 1, 1), jnp.float32)],
        scratch_shapes=[pltpu.VMEM((1, nb), jnp.float32),
                        pltpu.VMEM((1, nb), jnp.float32),
                        pltpu.VMEM((1, 1), jnp.float32)],
    )(a3, bt)

    ridx, rval, cidx = pl.pallas_call(
        _argmax_body,
        grid=(b, ni),
        in_specs=[a_spec, bt_spec, col_spec, row_spec, mean_spec, mean_spec],
        out_specs=[col_spec, col_spec, row_spec],
        out_shape=[jax.ShapeDtypeStruct((b * ni, BI, 1), jnp.int32),
                   jax.ShapeDtypeStruct((b * ni, BI, 1), jnp.float32),
                   jax.ShapeDtypeStruct((b, 1, nb), jnp.int32)],
        scratch_shapes=[pltpu.VMEM((1, nb), jnp.float32),
                        pltpu.VMEM((1, nb), jnp.int32)],
    )(a3, bt, alpha, beta, amean, bmean)

    max_idx_a = ridx.reshape(b, na)
    max_vals_a = rval.reshape(b, na)
    max_idx_b = cidx.reshape(b, nb)
    return max_idx_a, max_vals_a, max_idx_b


def _sc_epilogue(idx_a, vals_a, idx_b):
    b, na = idx_a.shape
    nb = idx_b.shape[1]
    info = plsc.get_sparse_core_info()
    nw = info.num_cores * info.num_subcores
    chunk = na // nw
    mesh = plsc.VectorSubcoreMesh(core_axis_name="c", subcore_axis_name="s")
    cp = pltpu.CompilerParams()
    if "needs_layout_passes" in pltpu.CompilerParams.__dataclass_fields__:
        cp = dataclasses.replace(cp, needs_layout_passes=False)

    @functools.partial(
        pl.kernel, mesh=mesh, compiler_params=cp,
        out_type=[jax.ShapeDtypeStruct((b, na), jnp.int32),
                  jax.ShapeDtypeStruct((b, nb), jnp.int32),
                  jax.ShapeDtypeStruct((b, na), jnp.float32),
                  jax.ShapeDtypeStruct((b, nb), jnp.float32)],
        scratch_types=[pltpu.VMEM((b * na,), jnp.int32),
                       pltpu.VMEM((b * nb,), jnp.int32),
                       pltpu.VMEM((b * na,), jnp.float32),
                       pltpu.VMEM((b, chunk), jnp.int32),
                       pltpu.VMEM((b, chunk), jnp.float32),
                       pltpu.VMEM((b, chunk), jnp.int32),
                       pltpu.VMEM((b, chunk), jnp.float32),
                       pltpu.SemaphoreType.DMA((b,)),
                       pltpu.SemaphoreType.DMA((b,))],
    )
    def epilogue(ia_hbm, va_hbm, ib_hbm, m0_hbm, m1_hbm, s0_hbm, s1_hbm,
                 ia_t, ib_t, va_t, m0_c, s0_c, m1_c, s1_c, insem, outsem):
        wid = lax.axis_index("s") * info.num_cores + lax.axis_index("c")
        base = wid * chunk
        in_copies = []
        for bb in range(b):
            in_copies.append((
                pltpu.async_copy(ia_hbm.at[bb], ia_t.at[pl.ds(bb * na, na)],
                                 insem.at[bb]),
                pltpu.async_copy(ib_hbm.at[bb], ib_t.at[pl.ds(bb * nb, nb)],
                                 insem.at[bb]),
                pltpu.async_copy(va_hbm.at[bb], va_t.at[pl.ds(bb * na, na)],
                                 insem.at[bb]),
            ))
        out_copies = []
        for bb in range(b):
            for cpy in in_copies[bb]:
                cpy.wait()

            @pl.loop(0, chunk, step=16)
            def _row(v):
                ja = ia_t[pl.ds(bb * na + base + v, 16)]
                back = plsc.load_gather(ib_t, [ja + bb * nb])
                rowid = lax.iota(jnp.int32, 16) + (base + v)
                val = va_t[pl.ds(bb * na + base + v, 16)]
                ok = (back == rowid) & (val > 0.0)
                m0_c[bb, pl.ds(v, 16)] = jnp.where(ok, ja, -1)
                s0_c[bb, pl.ds(v, 16)] = jnp.where(ok, val, 0.0)

            @pl.loop(0, chunk, step=16)
            def _col(v):
                jb = ib_t[pl.ds(bb * nb + base + v, 16)]
                fwd = plsc.load_gather(ia_t, [jb + bb * na])
                vb = plsc.load_gather(va_t, [jb + bb * na])
                colid = lax.iota(jnp.int32, 16) + (base + v)
                ok = (fwd == colid) & (vb > 0.0)
                m1_c[bb, pl.ds(v, 16)] = jnp.where(ok, jb, -1)
                s1_c[bb, pl.ds(v, 16)] = jnp.where(ok, vb, 0.0)

            out_copies.extend([
                pltpu.async_copy(m0_c.at[bb], m0_hbm.at[bb, pl.ds(base, chunk)],
                                 outsem.at[bb]),
                pltpu.async_copy(s0_c.at[bb], s0_hbm.at[bb, pl.ds(base, chunk)],
                                 outsem.at[bb]),
                pltpu.async_copy(m1_c.at[bb], m1_hbm.at[bb, pl.ds(base, chunk)],
                                 outsem.at[bb]),
                pltpu.async_copy(s1_c.at[bb], s1_hbm.at[bb, pl.ds(base, chunk)],
                                 outsem.at[bb]),
            ])
        for cpy in out_copies:
            cpy.wait()

    return epilogue(idx_a, vals_a, idx_b)


def kernel(keypoints_A, descriptions_A, keypoints_B, descriptions_B):
    max_idx_a, max_vals_a, max_idx_b = _tc_passes(descriptions_A, descriptions_B)
    matches0, matches1, scores0, scores1 = _sc_epilogue(
        max_idx_a, max_vals_a, max_idx_b)
    return matches0, matches1, scores0, scores1
